# baseline (speedup 1.0000x reference)
import functools

import jax
import jax.numpy as jnp
from jax import lax
from jax.experimental import pallas as pl
from jax.experimental.pallas import tpu as pltpu
from jax.experimental.pallas import tpu_sc as plsc

_B, _S, _H = 4, 8192, 1024
_N = _B * _S
_NC, _NS = 2, 16
_NW = _NC * _NS
_RPW = _N // _NW
_C = 32
_NCH = _RPW // _C
_NB = 3
_D = 1


def _build():
    mesh = plsc.VectorSubcoreMesh(core_axis_name="c", subcore_axis_name="s")

    @functools.partial(
        pl.kernel,
        mesh=mesh,
        out_type=[
            jax.ShapeDtypeStruct((_N, _H), jnp.float32),
            jax.ShapeDtypeStruct((_N, _H), jnp.float32),
        ],
        scratch_types=[
            pltpu.VMEM((_RPW,), jnp.int32),
            *[pltpu.VMEM((_C, _H), jnp.float32) for _ in range(_NB)],
            *[pltpu.SemaphoreType.DMA for _ in range(2 * _NB)],
        ],
    )
    def emb2(tids, fids, ttab, ftab, tout, fout, idx_v, *scratch):
        bufs = scratch[:_NB]
        gsems = scratch[_NB:2 * _NB]
        osems = scratch[2 * _NB:]
        wid = lax.axis_index("s") * _NC + lax.axis_index("c")
        base = wid * _RPW
        for ids_hbm, tab_hbm, out_hbm in ((tids, ttab, tout), (fids, ftab, fout)):
            pltpu.sync_copy(ids_hbm.at[pl.ds(base, _RPW)], idx_v)

            def gather_cp(g, b):
                return pltpu.make_async_copy(
                    tab_hbm.at[idx_v.at[pl.ds(g * _C, _C)]], bufs[b], gsems[b])

            def out_cp(g, b):
                return pltpu.make_async_copy(
                    bufs[b], out_hbm.at[pl.ds(base + g * _C, _C)], osems[b])

            def step(g, b):
                bm = (b - _D) % _NB
                out_cp(g - _NB, b).wait()
                gather_cp(g, b).start()
                gather_cp(g - _D, bm).wait()
                out_cp(g - _D, bm).start()

            for b in range(_NB):
                gather_cp(b, b).start()
            for k in range(_NB - _D):
                gather_cp(k, k).wait()
                out_cp(k, k).start()

            def body(j, carry):
                for b in range(_NB):
                    step(_NB * j + b, b)
                return carry

            lax.fori_loop(1, _NCH // _NB, body, 0)

            for g in range(_NB * (_NCH // _NB), _NCH):
                step(g, g % _NB)
            for g in range(_NCH - _D, _NCH):
                gather_cp(g, g % _NB).wait()
                out_cp(g, g % _NB).start()
            for g in range(_NCH - _NB, _NCH):
                out_cp(g, g % _NB).wait()

    return jax.jit(emb2)


_EMB2 = _build()


def kernel(input_ids, feature_ids, text_table, feature_table):
    tid = input_ids.reshape(-1).astype(jnp.int32)
    fid = feature_ids.reshape(-1).astype(jnp.int32)
    tout, fout = _EMB2(tid, fid, text_table, feature_table)
    return (tout.reshape(_B, _S, _H), fout.reshape(_B, _S, _H))


# speedup vs baseline: 1.0002x; 1.0002x over previous
import functools

import jax
import jax.numpy as jnp
from jax import lax
from jax.experimental import pallas as pl
from jax.experimental.pallas import tpu as pltpu
from jax.experimental.pallas import tpu_sc as plsc

_B, _S, _H = 4, 8192, 1024
_N = _B * _S
_NC, _NS = 2, 16
_NW = _NC * _NS
_RPW = _N // _NW
_C = 16
_NCH = _RPW // _C
_NB = 4
_D = 2


def _build():
    mesh = plsc.VectorSubcoreMesh(core_axis_name="c", subcore_axis_name="s")

    @functools.partial(
        pl.kernel,
        mesh=mesh,
        out_type=[
            jax.ShapeDtypeStruct((_N, _H), jnp.float32),
            jax.ShapeDtypeStruct((_N, _H), jnp.float32),
        ],
        scratch_types=[
            pltpu.VMEM((2 * _RPW,), jnp.int32),
            *[pltpu.VMEM((_C, _H), jnp.float32) for _ in range(_NB)],
            *[pltpu.SemaphoreType.DMA for _ in range(2 * _NB)],
        ],
    )
    def emb2(tids, fids, ttab, ftab, tout, fout, idx_v, *scratch):
        bufs = scratch[:_NB]
        gsems = scratch[_NB:2 * _NB]
        osems = scratch[2 * _NB:]
        wid = lax.axis_index("s") * _NC + lax.axis_index("c")
        base = wid * _RPW
        tabs = (ttab, ftab)
        outs = (tout, fout)

        def gather_cp(t, lc, b):
            return pltpu.make_async_copy(
                tabs[t].at[idx_v.at[pl.ds(t * _RPW + lc * _C, _C)]],
                bufs[b], gsems[b])

        def out_cp(t, lc, b):
            return pltpu.make_async_copy(
                bufs[b], outs[t].at[pl.ds(base + lc * _C, _C)], osems[b])

        pltpu.sync_copy(tids.at[pl.ds(base, _RPW)], idx_v.at[pl.ds(0, _RPW)])
        for b in range(_NB):
            gather_cp(0, b, b).start()
        pltpu.sync_copy(fids.at[pl.ds(base, _RPW)], idx_v.at[pl.ds(_RPW, _RPW)])
        for k in range(_NB - _D):
            gather_cp(0, k, k).wait()
            out_cp(0, k, k).start()

        def mk_body(t):
            def body(j, carry):
                for b in range(_NB):
                    lc = _NB * j + b
                    bm = (b - _D) % _NB
                    out_cp(t, lc - _NB, b).wait()
                    gather_cp(t, lc, b).start()
                    gather_cp(t, lc - _D, bm).wait()
                    out_cp(t, lc - _D, bm).start()
                return carry
            return body

        lax.fori_loop(1, _NCH // _NB, mk_body(0), 0)

        for c in range(_NCH, _NCH + _NB):
            b = c % _NB
            bm = (b - _D) % _NB
            cd = c - _D
            out_cp(0, c - _NB, b).wait()
            gather_cp(1, c - _NCH, b).start()
            gather_cp(cd // _NCH, cd % _NCH, bm).wait()
            out_cp(cd // _NCH, cd % _NCH, bm).start()

        lax.fori_loop(1, _NCH // _NB, mk_body(1), 0)

        for lc in range(_NCH - _D, _NCH):
            gather_cp(1, lc, lc % _NB).wait()
            out_cp(1, lc, lc % _NB).start()
        for lc in range(_NCH - _NB, _NCH):
            out_cp(1, lc, lc % _NB).wait()

    return jax.jit(emb2)


_EMB2 = _build()


def kernel(input_ids, feature_ids, text_table, feature_table):
    tid = input_ids.reshape(-1).astype(jnp.int32)
    fid = feature_ids.reshape(-1).astype(jnp.int32)
    tout, fout = _EMB2(tid, fid, text_table, feature_table)
    return (tout.reshape(_B, _S, _H), fout.reshape(_B, _S, _H))
